# Initial kernel scaffold; baseline (speedup 1.0000x reference)
import jax
import jax.numpy as jnp
from jax.experimental import pallas as pl


def kernel(x, edge_index, batch, W1, a_s1, a_d1, b1, W2, a_s2, a_d2, b2, aw, ab, fc1_w, fc1_b, bn_g, bn_b, fc2_w, fc2_b):
    raise NotImplementedError("write your pallas kernel here")


# speedup vs baseline: 17.9409x; 17.9409x over previous
import functools

import jax
import jax.numpy as jnp
from jax import lax
from jax.experimental import pallas as pl
from jax.experimental.pallas import tpu as pltpu
from jax.experimental.pallas import tpu_sc as plsc

N = 10000
E = 320000
D = 128
G = 256
NC = 2
NS = 16
LANES = 16
NW = NC * NS
CH = 128
KCH = 80
EPW = CH * KCH
EPAD = EPW * NW
ACC_R = 10240
STRIPE = ACC_R // NS
TAB = N + 16


def _leaky(x, s):
    return jnp.where(x >= 0, x, x * s)


def _tc1_body(x_ref, w_ref, asv_ref, adv_ref, h_ref, s_ref, d_ref):
    h = jnp.dot(x_ref[...], w_ref[...], preferred_element_type=jnp.float32)
    h_ref[...] = h
    s_ref[...] = jnp.sum(h * asv_ref[...], axis=1, keepdims=True)
    d_ref[...] = jnp.sum(h * adv_ref[...], axis=1, keepdims=True)


def _tc1(x, W, asv, adv):
    R = 2000
    return pl.pallas_call(
        _tc1_body,
        grid=(N // R,),
        in_specs=[
            pl.BlockSpec((R, D), lambda i: (i, 0)),
            pl.BlockSpec((D, D), lambda i: (0, 0)),
            pl.BlockSpec((1, D), lambda i: (0, 0)),
            pl.BlockSpec((1, D), lambda i: (0, 0)),
        ],
        out_specs=[
            pl.BlockSpec((R, D), lambda i: (i, 0)),
            pl.BlockSpec((R, 1), lambda i: (i, 0)),
            pl.BlockSpec((R, 1), lambda i: (i, 0)),
        ],
        out_shape=[
            jax.ShapeDtypeStruct((N, D), jnp.float32),
            jax.ShapeDtypeStruct((N, 1), jnp.float32),
            jax.ShapeDtypeStruct((N, 1), jnp.float32),
        ],
    )(x, W, asv, adv)


def _combine(acc0, acc1, den0, den1, h, s_col, d_col, b):
    e = _leaky(s_col + d_col, 0.2)
    exs = jnp.exp(e)
    num = acc0 + acc1 + exs * h
    den = den0 + den1 + exs + 1e-16
    return _leaky(num / den + b, 0.01)


def _tc2_body(acc0_ref, acc1_ref, den0_ref, den1_ref, h_ref, s_ref, d_ref,
              b_ref, w_ref, asv_ref, adv_ref, h2_ref, s2_ref, d2_ref):
    x2 = _combine(acc0_ref[...], acc1_ref[...], den0_ref[...], den1_ref[...],
                  h_ref[...], s_ref[...], d_ref[...], b_ref[...])
    h2 = jnp.dot(x2, w_ref[...], preferred_element_type=jnp.float32)
    h2_ref[...] = h2
    s2_ref[...] = jnp.sum(h2 * asv_ref[...], axis=1, keepdims=True)
    d2_ref[...] = jnp.sum(h2 * adv_ref[...], axis=1, keepdims=True)


def _tc2(acc0, acc1, den0, den1, h, s_col, d_col, b, W, asv, adv):
    R = 2000
    col = pl.BlockSpec((R, 1), lambda i: (i, 0))
    mat = pl.BlockSpec((R, D), lambda i: (i, 0))
    one = pl.BlockSpec((1, D), lambda i: (0, 0))
    return pl.pallas_call(
        _tc2_body,
        grid=(N // R,),
        in_specs=[mat, mat, col, col, mat, col, col, one,
                  pl.BlockSpec((D, D), lambda i: (0, 0)), one, one],
        out_specs=[mat, col, col],
        out_shape=[
            jax.ShapeDtypeStruct((N, D), jnp.float32),
            jax.ShapeDtypeStruct((N, 1), jnp.float32),
            jax.ShapeDtypeStruct((N, 1), jnp.float32),
        ],
    )(acc0, acc1, den0, den1, h, s_col, d_col, b, W, asv, adv)


def _tc3a_body(acc0_ref, acc1_ref, den0_ref, den1_ref, h_ref, s_ref, d_ref,
               b_ref, aw_ref, ab_ref, hf_ref, sc_ref):
    x3 = _combine(acc0_ref[...], acc1_ref[...], den0_ref[...], den1_ref[...],
                  h_ref[...], s_ref[...], d_ref[...], b_ref[...])
    hf_ref[...] = x3
    sc_ref[...] = jnp.dot(x3, aw_ref[...],
                          preferred_element_type=jnp.float32) + ab_ref[...]


def _tc3a(acc0, acc1, den0, den1, h, s_col, d_col, b, aw, ab):
    R = 2000
    col = pl.BlockSpec((R, 1), lambda i: (i, 0))
    mat = pl.BlockSpec((R, D), lambda i: (i, 0))
    return pl.pallas_call(
        _tc3a_body,
        grid=(N // R,),
        in_specs=[mat, mat, col, col, mat, col, col,
                  pl.BlockSpec((1, D), lambda i: (0, 0)),
                  pl.BlockSpec((D, 1), lambda i: (0, 0)),
                  pl.BlockSpec((1, 1), lambda i: (0, 0))],
        out_specs=[mat, col],
        out_shape=[
            jax.ShapeDtypeStruct((N, D), jnp.float32),
            jax.ShapeDtypeStruct((N, 1), jnp.float32),
        ],
    )(acc0, acc1, den0, den1, h, s_col, d_col, b, aw, ab)


def _tc3b_body(s_ref, attn_ref):
    s = s_ref[...]
    m = jnp.max(s)
    p = jnp.exp(s - m)
    attn_ref[...] = p / jnp.sum(p)


def _tc3b(s_col):
    return pl.pallas_call(
        _tc3b_body,
        out_shape=jax.ShapeDtypeStruct((N, 1), jnp.float32),
    )(s_col)


def _tc3c_body(hf_ref, attn_ref, batch_ref, f1w_ref, f1b_ref, bng_ref,
               bnb_ref, f2w_ref, f2b_ref, z_ref, g_acc):
    i = pl.program_id(0)

    @pl.when(i == 0)
    def _():
        g_acc[...] = jnp.zeros_like(g_acc)

    gid = lax.broadcasted_iota(jnp.int32, (1, G), 1)
    onehot = (batch_ref[...] == gid).astype(jnp.float32)
    w = attn_ref[...] * hf_ref[...]
    g_acc[...] += lax.dot_general(onehot, w, (((0,), (0,)), ((), ())),
                                  preferred_element_type=jnp.float32)

    @pl.when(i == pl.num_programs(0) - 1)
    def _():
        g = g_acc[...]
        z = jnp.dot(g, f1w_ref[...],
                    preferred_element_type=jnp.float32) + f1b_ref[...]
        mean = jnp.mean(z, axis=0, keepdims=True)
        var = jnp.mean((z - mean) * (z - mean), axis=0, keepdims=True)
        z = bng_ref[...] * (z - mean) / jnp.sqrt(var + 1e-5) + bnb_ref[...]
        z = _leaky(z, 0.01)
        z_ref[...] = jnp.dot(z, f2w_ref[...],
                             preferred_element_type=jnp.float32) + f2b_ref[...]


def _tc3c(hf, attn, batch_col, f1w, f1b, bng, bnb, f2w, f2b):
    R = 2000
    H = D // 2
    return pl.pallas_call(
        _tc3c_body,
        grid=(N // R,),
        in_specs=[
            pl.BlockSpec((R, D), lambda i: (i, 0)),
            pl.BlockSpec((R, 1), lambda i: (i, 0)),
            pl.BlockSpec((R, 1), lambda i: (i, 0)),
            pl.BlockSpec((D, H), lambda i: (0, 0)),
            pl.BlockSpec((1, H), lambda i: (0, 0)),
            pl.BlockSpec((1, H), lambda i: (0, 0)),
            pl.BlockSpec((1, H), lambda i: (0, 0)),
            pl.BlockSpec((H, D), lambda i: (0, 0)),
            pl.BlockSpec((1, D), lambda i: (0, 0)),
        ],
        out_specs=pl.BlockSpec((G, D), lambda i: (0, 0)),
        out_shape=jax.ShapeDtypeStruct((G, D), jnp.float32),
        scratch_shapes=[pltpu.VMEM((G, D), jnp.float32)],
    )(hf, attn, batch_col, f1w, f1b, bng, bnb, f2w, f2b)


BLK = 8
NBLK = KCH // BLK


def _make_sc_mesh():
    return plsc.VectorSubcoreMesh(core_axis_name="c", subcore_axis_name="s",
                                  num_cores=NC, num_subcores=NS)


def _make_sc_ex():
    @functools.partial(
        pl.kernel,
        out_type=[
            jax.ShapeDtypeStruct((EPAD,), jnp.float32),
            jax.ShapeDtypeStruct((NC * ACC_R,), jnp.float32),
        ],
        mesh=_make_sc_mesh(),
        compiler_params=pltpu.CompilerParams(needs_layout_passes=False),
        scratch_types=[
            pltpu.VMEM_SHARED((ACC_R,), jnp.float32),
            pltpu.VMEM((TAB,), jnp.float32),
            pltpu.VMEM((TAB,), jnp.float32),
            pltpu.VMEM((BLK, CH), jnp.int32),
            pltpu.VMEM((BLK, CH), jnp.int32),
            pltpu.VMEM((CH,), jnp.float32),
            pltpu.VMEM((STRIPE,), jnp.float32),
        ],
    )
    def sc_ex(src_hbm, dst_hbm, as_hbm, ad_hbm, ex_out, den_out,
              den_sh, as_tab, ad_tab, srcb, dstb, exb, denb):
        c = lax.axis_index("c")
        s = lax.axis_index("s")
        w = c * NS + s
        z16 = jnp.zeros((LANES,), jnp.float32)

        pltpu.sync_copy(as_hbm, as_tab)
        pltpu.sync_copy(ad_hbm, ad_tab)

        def _zden(i, _):
            denb[pl.ds(i * LANES, LANES)] = z16
            return 0
        lax.fori_loop(0, STRIPE // LANES, _zden, 0)
        pltpu.sync_copy(denb, den_sh.at[pl.ds(s * STRIPE, STRIPE)])
        plsc.subcore_barrier()

        def _blk(b, _):
            pltpu.sync_copy(src_hbm.at[w, pl.ds(b * BLK, BLK)], srcb)
            pltpu.sync_copy(dst_hbm.at[w, pl.ds(b * BLK, BLK)], dstb)

            def _chunk(jj, _):
                def _group(g, _):
                    src16 = srcb[jj, pl.ds(g * LANES, LANES)]
                    dst16 = dstb[jj, pl.ds(g * LANES, LANES)]
                    e = (plsc.load_gather(as_tab, [src16])
                         + plsc.load_gather(ad_tab, [dst16]))
                    e = jnp.where(e >= 0, e, e * 0.2)
                    exb[pl.ds(g * LANES, LANES)] = jnp.exp(e)
                    return 0
                lax.fori_loop(0, CH // LANES, _group, 0)
                off = (w * KCH + b * BLK + jj) * CH
                pltpu.sync_copy(exb, ex_out.at[pl.ds(off, CH)])
                pltpu.sync_copy(exb, den_sh.at[dstb.at[jj]], add=True)
                return 0
            lax.fori_loop(0, BLK, _chunk, 0)
            return 0
        lax.fori_loop(0, NBLK, _blk, 0)

        plsc.subcore_barrier()
        pltpu.sync_copy(den_sh.at[pl.ds(s * STRIPE, STRIPE)], denb)
        pltpu.sync_copy(denb, den_out.at[pl.ds(c * ACC_R + s * STRIPE,
                                               STRIPE)])

    return sc_ex


def _make_sc_rows():
    @functools.partial(
        pl.kernel,
        out_type=jax.ShapeDtypeStruct((NC * ACC_R, D), jnp.float32),
        mesh=_make_sc_mesh(),
        compiler_params=pltpu.CompilerParams(needs_layout_passes=False),
        scratch_types=[
            pltpu.VMEM_SHARED((ACC_R, D), jnp.float32),
            pltpu.VMEM((BLK, CH), jnp.int32),
            pltpu.VMEM((BLK, CH), jnp.int32),
            pltpu.VMEM((BLK * CH,), jnp.float32),
            pltpu.VMEM((CH, D), jnp.float32),
            pltpu.SemaphoreType.DMA,
        ],
    )
    def sc_rows(h_hbm, src_hbm, dst_hbm, ex_hbm, acc_out,
                acc_sh, srcb, dstb, exbuf, rows, gsem):
        c = lax.axis_index("c")
        s = lax.axis_index("s")
        w = c * NS + s
        z16 = jnp.zeros((LANES,), jnp.float32)

        def _zrow(r, _):
            for cc in range(D // LANES):
                rows[r, pl.ds(cc * LANES, LANES)] = z16
            return 0
        lax.fori_loop(0, CH, _zrow, 0)
        for k in range(STRIPE // CH):
            pltpu.sync_copy(rows, acc_sh.at[pl.ds(s * STRIPE + k * CH, CH)])
        plsc.subcore_barrier()

        def _blk(b, _):
            pltpu.sync_copy(src_hbm.at[w, pl.ds(b * BLK, BLK)], srcb)
            pltpu.sync_copy(dst_hbm.at[w, pl.ds(b * BLK, BLK)], dstb)
            pltpu.sync_copy(
                ex_hbm.at[pl.ds((w * KCH + b * BLK) * CH, BLK * CH)], exbuf)

            def _chunk(jj, _):
                pltpu.async_copy(h_hbm.at[srcb.at[jj]], rows, gsem).wait()

                def _group(g, _):
                    ex16 = exbuf[pl.ds(jj * CH + g * LANES, LANES)]
                    for ll in range(LANES):
                        r = g * LANES + ll
                        sv = ex16[ll]
                        for cc in range(D // LANES):
                            sl = pl.ds(cc * LANES, LANES)
                            rows[r, sl] = rows[r, sl] * sv
                    return 0
                lax.fori_loop(0, CH // LANES, _group, 0)

                pltpu.sync_copy(rows, acc_sh.at[dstb.at[jj]], add=True)
                return 0
            lax.fori_loop(0, BLK, _chunk, 0)
            return 0
        lax.fori_loop(0, NBLK, _blk, 0)

        plsc.subcore_barrier()

        def _cp(k, _):
            off = s * STRIPE + k * CH
            pltpu.sync_copy(acc_sh.at[pl.ds(off, CH)], rows)
            pltpu.sync_copy(rows, acc_out.at[pl.ds(c * ACC_R + off, CH)])
            return 0
        lax.fori_loop(0, STRIPE // CH, _cp, 0)

    return sc_rows


_sc_kernels = None


def _get_sc_kernels():
    global _sc_kernels
    if _sc_kernels is None:
        _sc_kernels = (_make_sc_ex(), _make_sc_rows())
    return _sc_kernels


def kernel(x, edge_index, batch, W1, a_s1, a_d1, b1, W2, a_s2, a_d2, b2, aw,
           ab, fc1_w, fc1_b, bn_g, bn_b, fc2_w, fc2_b):
    f32 = jnp.float32
    src = edge_index[0].astype(jnp.int32)
    dst = edge_index[1].astype(jnp.int32)
    pad = EPAD - E
    src_p = jnp.concatenate([src, jnp.zeros((pad,), jnp.int32)])
    dst_p = jnp.concatenate([dst, jnp.full((pad,), N, jnp.int32)])
    src2d = src_p.reshape(NW, KCH, CH)
    dst2d = dst_p.reshape(NW, KCH, CH)

    as1r = a_s1.reshape(1, D).astype(f32)
    ad1r = a_d1.reshape(1, D).astype(f32)
    as2r = a_s2.reshape(1, D).astype(f32)
    ad2r = a_d2.reshape(1, D).astype(f32)
    b1r = b1.reshape(1, D).astype(f32)
    b2r = b2.reshape(1, D).astype(f32)

    sc_ex, sc_rows = _get_sc_kernels()

    h1, s1, d1 = _tc1(x.astype(f32), W1.astype(f32), as1r, ad1r)
    as_pad = jnp.concatenate([s1.reshape(N), jnp.zeros((TAB - N,), f32)])
    ad_pad = jnp.concatenate([d1.reshape(N), jnp.zeros((TAB - N,), f32)])
    ex1, den = sc_ex(src2d, dst2d, as_pad, ad_pad)
    acc = sc_rows(h1, src2d, dst2d, ex1)
    acc0, acc1 = acc[:N], acc[ACC_R:ACC_R + N]
    den0 = den[:N].reshape(N, 1)
    den1 = den[ACC_R:ACC_R + N].reshape(N, 1)

    h2, s2, d2 = _tc2(acc0, acc1, den0, den1, h1, s1, d1, b1r,
                      W2.astype(f32), as2r, ad2r)
    as_pad2 = jnp.concatenate([s2.reshape(N), jnp.zeros((TAB - N,), f32)])
    ad_pad2 = jnp.concatenate([d2.reshape(N), jnp.zeros((TAB - N,), f32)])
    ex2, denx = sc_ex(src2d, dst2d, as_pad2, ad_pad2)
    accb = sc_rows(h2, src2d, dst2d, ex2)
    acc0b, acc1b = accb[:N], accb[ACC_R:ACC_R + N]
    den0b = denx[:N].reshape(N, 1)
    den1b = denx[ACC_R:ACC_R + N].reshape(N, 1)

    hf, s_col = _tc3a(acc0b, acc1b, den0b, den1b, h2, s2, d2, b2r,
                      aw.reshape(D, 1).astype(f32),
                      ab.reshape(1, 1).astype(f32))
    attn = _tc3b(s_col)
    batch_col = batch.astype(jnp.int32).reshape(N, 1)
    z = _tc3c(hf, attn, batch_col,
              fc1_w.astype(f32), fc1_b.reshape(1, -1).astype(f32),
              bn_g.reshape(1, -1).astype(f32), bn_b.reshape(1, -1).astype(f32),
              fc2_w.astype(f32), fc2_b.reshape(1, -1).astype(f32))
    return z


# speedup vs baseline: 32.3664x; 1.8041x over previous
import functools

import jax
import jax.numpy as jnp
from jax import lax
from jax.experimental import pallas as pl
from jax.experimental.pallas import tpu as pltpu
from jax.experimental.pallas import tpu_sc as plsc

N = 10000
E = 320000
D = 128
G = 256
NC = 2
NS = 16
LANES = 16
NW = NC * NS
CH = 128
KCH = 80
EPW = CH * KCH
EPAD = EPW * NW
ACC_R = 10240
STRIPE = ACC_R // NS
TAB = N + 16


def _leaky(x, s):
    return jnp.where(x >= 0, x, x * s)


def _tc1_body(x_ref, w_ref, asv_ref, adv_ref, h_ref, s_ref, d_ref):
    h = jnp.dot(x_ref[...], w_ref[...], preferred_element_type=jnp.float32)
    h_ref[...] = h
    s_ref[...] = jnp.sum(h * asv_ref[...], axis=1, keepdims=True)
    d_ref[...] = jnp.sum(h * adv_ref[...], axis=1, keepdims=True)


def _tc1(x, W, asv, adv):
    R = 2000
    return pl.pallas_call(
        _tc1_body,
        grid=(N // R,),
        in_specs=[
            pl.BlockSpec((R, D), lambda i: (i, 0)),
            pl.BlockSpec((D, D), lambda i: (0, 0)),
            pl.BlockSpec((1, D), lambda i: (0, 0)),
            pl.BlockSpec((1, D), lambda i: (0, 0)),
        ],
        out_specs=[
            pl.BlockSpec((R, D), lambda i: (i, 0)),
            pl.BlockSpec((R, 1), lambda i: (i, 0)),
            pl.BlockSpec((R, 1), lambda i: (i, 0)),
        ],
        out_shape=[
            jax.ShapeDtypeStruct((N, D), jnp.float32),
            jax.ShapeDtypeStruct((N, 1), jnp.float32),
            jax.ShapeDtypeStruct((N, 1), jnp.float32),
        ],
    )(x, W, asv, adv)


def _combine(acc0, acc1, den0, den1, h, s_col, d_col, b):
    e = _leaky(s_col + d_col, 0.2)
    exs = jnp.exp(e)
    num = acc0 + acc1 + exs * h
    den = den0 + den1 + exs + 1e-16
    return _leaky(num / den + b, 0.01)


def _tc2_body(acc0_ref, acc1_ref, den0_ref, den1_ref, h_ref, s_ref, d_ref,
              b_ref, w_ref, asv_ref, adv_ref, h2_ref, s2_ref, d2_ref):
    x2 = _combine(acc0_ref[...], acc1_ref[...], den0_ref[...], den1_ref[...],
                  h_ref[...], s_ref[...], d_ref[...], b_ref[...])
    h2 = jnp.dot(x2, w_ref[...], preferred_element_type=jnp.float32)
    h2_ref[...] = h2
    s2_ref[...] = jnp.sum(h2 * asv_ref[...], axis=1, keepdims=True)
    d2_ref[...] = jnp.sum(h2 * adv_ref[...], axis=1, keepdims=True)


def _tc2(acc0, acc1, den0, den1, h, s_col, d_col, b, W, asv, adv):
    R = 2000
    col = pl.BlockSpec((R, 1), lambda i: (i, 0))
    mat = pl.BlockSpec((R, D), lambda i: (i, 0))
    one = pl.BlockSpec((1, D), lambda i: (0, 0))
    return pl.pallas_call(
        _tc2_body,
        grid=(N // R,),
        in_specs=[mat, mat, col, col, mat, col, col, one,
                  pl.BlockSpec((D, D), lambda i: (0, 0)), one, one],
        out_specs=[mat, col, col],
        out_shape=[
            jax.ShapeDtypeStruct((N, D), jnp.float32),
            jax.ShapeDtypeStruct((N, 1), jnp.float32),
            jax.ShapeDtypeStruct((N, 1), jnp.float32),
        ],
    )(acc0, acc1, den0, den1, h, s_col, d_col, b, W, asv, adv)


def _tc3a_body(acc0_ref, acc1_ref, den0_ref, den1_ref, h_ref, s_ref, d_ref,
               b_ref, aw_ref, ab_ref, hf_ref, sc_ref):
    x3 = _combine(acc0_ref[...], acc1_ref[...], den0_ref[...], den1_ref[...],
                  h_ref[...], s_ref[...], d_ref[...], b_ref[...])
    hf_ref[...] = x3
    sc_ref[...] = jnp.dot(x3, aw_ref[...],
                          preferred_element_type=jnp.float32) + ab_ref[...]


def _tc3a(acc0, acc1, den0, den1, h, s_col, d_col, b, aw, ab):
    R = 2000
    col = pl.BlockSpec((R, 1), lambda i: (i, 0))
    mat = pl.BlockSpec((R, D), lambda i: (i, 0))
    return pl.pallas_call(
        _tc3a_body,
        grid=(N // R,),
        in_specs=[mat, mat, col, col, mat, col, col,
                  pl.BlockSpec((1, D), lambda i: (0, 0)),
                  pl.BlockSpec((D, 1), lambda i: (0, 0)),
                  pl.BlockSpec((1, 1), lambda i: (0, 0))],
        out_specs=[mat, col],
        out_shape=[
            jax.ShapeDtypeStruct((N, D), jnp.float32),
            jax.ShapeDtypeStruct((N, 1), jnp.float32),
        ],
    )(acc0, acc1, den0, den1, h, s_col, d_col, b, aw, ab)


def _tc3b_body(s_ref, attn_ref):
    s = s_ref[...]
    m = jnp.max(s)
    p = jnp.exp(s - m)
    attn_ref[...] = p / jnp.sum(p)


def _tc3b(s_col):
    return pl.pallas_call(
        _tc3b_body,
        out_shape=jax.ShapeDtypeStruct((N, 1), jnp.float32),
    )(s_col)


def _tc3c_body(hf_ref, attn_ref, batch_ref, f1w_ref, f1b_ref, bng_ref,
               bnb_ref, f2w_ref, f2b_ref, z_ref, g_acc):
    i = pl.program_id(0)

    @pl.when(i == 0)
    def _():
        g_acc[...] = jnp.zeros_like(g_acc)

    gid = lax.broadcasted_iota(jnp.int32, (1, G), 1)
    onehot = (batch_ref[...] == gid).astype(jnp.float32)
    w = attn_ref[...] * hf_ref[...]
    g_acc[...] += lax.dot_general(onehot, w, (((0,), (0,)), ((), ())),
                                  preferred_element_type=jnp.float32)

    @pl.when(i == pl.num_programs(0) - 1)
    def _():
        g = g_acc[...]
        z = jnp.dot(g, f1w_ref[...],
                    preferred_element_type=jnp.float32) + f1b_ref[...]
        mean = jnp.mean(z, axis=0, keepdims=True)
        var = jnp.mean((z - mean) * (z - mean), axis=0, keepdims=True)
        z = bng_ref[...] * (z - mean) / jnp.sqrt(var + 1e-5) + bnb_ref[...]
        z = _leaky(z, 0.01)
        z_ref[...] = jnp.dot(z, f2w_ref[...],
                             preferred_element_type=jnp.float32) + f2b_ref[...]


def _tc3c(hf, attn, batch_col, f1w, f1b, bng, bnb, f2w, f2b):
    R = 2000
    H = D // 2
    return pl.pallas_call(
        _tc3c_body,
        grid=(N // R,),
        in_specs=[
            pl.BlockSpec((R, D), lambda i: (i, 0)),
            pl.BlockSpec((R, 1), lambda i: (i, 0)),
            pl.BlockSpec((R, 1), lambda i: (i, 0)),
            pl.BlockSpec((D, H), lambda i: (0, 0)),
            pl.BlockSpec((1, H), lambda i: (0, 0)),
            pl.BlockSpec((1, H), lambda i: (0, 0)),
            pl.BlockSpec((1, H), lambda i: (0, 0)),
            pl.BlockSpec((H, D), lambda i: (0, 0)),
            pl.BlockSpec((1, D), lambda i: (0, 0)),
        ],
        out_specs=pl.BlockSpec((G, D), lambda i: (0, 0)),
        out_shape=jax.ShapeDtypeStruct((G, D), jnp.float32),
        scratch_shapes=[pltpu.VMEM((G, D), jnp.float32)],
    )(hf, attn, batch_col, f1w, f1b, bng, bnb, f2w, f2b)


BLK = 8
NBLK = KCH // BLK


def _make_sc_mesh():
    return plsc.VectorSubcoreMesh(core_axis_name="c", subcore_axis_name="s",
                                  num_cores=NC, num_subcores=NS)


def _make_sc_ex():
    @functools.partial(
        pl.kernel,
        out_type=[
            jax.ShapeDtypeStruct((EPAD,), jnp.float32),
            jax.ShapeDtypeStruct((NC * ACC_R,), jnp.float32),
        ],
        mesh=_make_sc_mesh(),
        compiler_params=pltpu.CompilerParams(needs_layout_passes=False),
        scratch_types=[
            pltpu.VMEM_SHARED((ACC_R,), jnp.float32),
            pltpu.VMEM((TAB,), jnp.float32),
            pltpu.VMEM((TAB,), jnp.float32),
            pltpu.VMEM((BLK, CH), jnp.int32),
            pltpu.VMEM((BLK, CH), jnp.int32),
            pltpu.VMEM((CH,), jnp.float32),
            pltpu.VMEM((STRIPE,), jnp.float32),
        ],
    )
    def sc_ex(src_hbm, dst_hbm, as_hbm, ad_hbm, ex_out, den_out,
              den_sh, as_tab, ad_tab, srcb, dstb, exb, denb):
        c = lax.axis_index("c")
        s = lax.axis_index("s")
        w = c * NS + s
        z16 = jnp.zeros((LANES,), jnp.float32)

        pltpu.sync_copy(as_hbm, as_tab)
        pltpu.sync_copy(ad_hbm, ad_tab)

        def _zden(i, _):
            denb[pl.ds(i * LANES, LANES)] = z16
            return 0
        lax.fori_loop(0, STRIPE // LANES, _zden, 0)
        pltpu.sync_copy(denb, den_sh.at[pl.ds(s * STRIPE, STRIPE)])
        plsc.subcore_barrier()

        n_real = jnp.clip(E // CH - w * KCH, 0, KCH)
        nblk = (n_real + BLK - 1) // BLK

        def _blk(b, _):
            pltpu.sync_copy(src_hbm.at[w, pl.ds(b * BLK, BLK)], srcb)
            pltpu.sync_copy(dst_hbm.at[w, pl.ds(b * BLK, BLK)], dstb)

            def _chunk(jj, _):
                def _group(g, _):
                    src16 = srcb[jj, pl.ds(g * LANES, LANES)]
                    dst16 = dstb[jj, pl.ds(g * LANES, LANES)]
                    e = (plsc.load_gather(as_tab, [src16])
                         + plsc.load_gather(ad_tab, [dst16]))
                    e = jnp.where(e >= 0, e, e * 0.2)
                    exb[pl.ds(g * LANES, LANES)] = jnp.exp(e)
                    return 0
                lax.fori_loop(0, CH // LANES, _group, 0)
                off = (w * KCH + b * BLK + jj) * CH
                pltpu.sync_copy(exb, ex_out.at[pl.ds(off, CH)])
                pltpu.sync_copy(exb, den_sh.at[dstb.at[jj]], add=True)
                return 0
            lax.fori_loop(0, jnp.minimum(BLK, n_real - b * BLK), _chunk, 0)
            return 0
        lax.fori_loop(0, nblk, _blk, 0)

        plsc.subcore_barrier()
        pltpu.sync_copy(den_sh.at[pl.ds(s * STRIPE, STRIPE)], denb)
        pltpu.sync_copy(denb, den_out.at[pl.ds(c * ACC_R + s * STRIPE,
                                               STRIPE)])

    return sc_ex


def _make_sc_rows():
    @functools.partial(
        pl.kernel,
        out_type=jax.ShapeDtypeStruct((NC * ACC_R, D), jnp.float32),
        mesh=_make_sc_mesh(),
        compiler_params=pltpu.CompilerParams(needs_layout_passes=False),
        scratch_types=[
            pltpu.VMEM_SHARED((ACC_R, D), jnp.float32),
            pltpu.VMEM((BLK, CH), jnp.int32),
            pltpu.VMEM((BLK, CH), jnp.int32),
            pltpu.VMEM((BLK * CH,), jnp.float32),
            pltpu.VMEM((CH, D), jnp.float32),
            pltpu.SemaphoreType.DMA,
        ],
    )
    def sc_rows(h_hbm, src_hbm, dst_hbm, ex_hbm, acc_out,
                acc_sh, srcb, dstb, exbuf, rows, gsem):
        c = lax.axis_index("c")
        s = lax.axis_index("s")
        w = c * NS + s
        z16 = jnp.zeros((LANES,), jnp.float32)

        def _zrow(r, _):
            for cc in range(D // LANES):
                rows[r, pl.ds(cc * LANES, LANES)] = z16
            return 0
        lax.fori_loop(0, CH, _zrow, 0)
        for k in range(STRIPE // CH):
            pltpu.sync_copy(rows, acc_sh.at[pl.ds(s * STRIPE + k * CH, CH)])
        plsc.subcore_barrier()

        n_real = jnp.clip(E // CH - w * KCH, 0, KCH)
        nblk = (n_real + BLK - 1) // BLK

        def _blk(b, _):
            pltpu.sync_copy(src_hbm.at[w, pl.ds(b * BLK, BLK)], srcb)
            pltpu.sync_copy(dst_hbm.at[w, pl.ds(b * BLK, BLK)], dstb)
            pltpu.sync_copy(
                ex_hbm.at[pl.ds((w * KCH + b * BLK) * CH, BLK * CH)], exbuf)

            def _chunk(jj, _):
                pltpu.async_copy(h_hbm.at[srcb.at[jj]], rows, gsem).wait()

                def _group(g, _):
                    ex16 = exbuf[pl.ds(jj * CH + g * LANES, LANES)]
                    for ll in range(LANES):
                        r = g * LANES + ll
                        sv = ex16[ll]
                        for cc in range(D // LANES):
                            sl = pl.ds(cc * LANES, LANES)
                            rows[r, sl] = rows[r, sl] * sv
                    return 0
                lax.fori_loop(0, CH // LANES, _group, 0)

                pltpu.sync_copy(rows, acc_sh.at[dstb.at[jj]], add=True)
                return 0
            lax.fori_loop(0, jnp.minimum(BLK, n_real - b * BLK), _chunk, 0)
            return 0
        lax.fori_loop(0, nblk, _blk, 0)

        plsc.subcore_barrier()

        def _cp(k, _):
            off = s * STRIPE + k * CH
            pltpu.sync_copy(acc_sh.at[pl.ds(off, CH)], rows)
            pltpu.sync_copy(rows, acc_out.at[pl.ds(c * ACC_R + off, CH)])
            return 0
        lax.fori_loop(0, STRIPE // CH, _cp, 0)

    return sc_rows


_sc_kernels = None


def _get_sc_kernels():
    global _sc_kernels
    if _sc_kernels is None:
        _sc_kernels = (_make_sc_ex(), _make_sc_rows())
    return _sc_kernels


def kernel(x, edge_index, batch, W1, a_s1, a_d1, b1, W2, a_s2, a_d2, b2, aw,
           ab, fc1_w, fc1_b, bn_g, bn_b, fc2_w, fc2_b):
    f32 = jnp.float32
    src = edge_index[0].astype(jnp.int32)
    dst = edge_index[1].astype(jnp.int32)
    pad = EPAD - E
    src_p = jnp.concatenate([src, jnp.zeros((pad,), jnp.int32)])
    dst_p = jnp.concatenate([dst, jnp.full((pad,), N, jnp.int32)])
    src2d = src_p.reshape(NW, KCH, CH)
    dst2d = dst_p.reshape(NW, KCH, CH)

    as1r = a_s1.reshape(1, D).astype(f32)
    ad1r = a_d1.reshape(1, D).astype(f32)
    as2r = a_s2.reshape(1, D).astype(f32)
    ad2r = a_d2.reshape(1, D).astype(f32)
    b1r = b1.reshape(1, D).astype(f32)
    b2r = b2.reshape(1, D).astype(f32)

    sc_ex, sc_rows = _get_sc_kernels()

    h1, s1, d1 = _tc1(x.astype(f32), W1.astype(f32), as1r, ad1r)
    as_pad = jnp.concatenate([s1.reshape(N), jnp.zeros((TAB - N,), f32)])
    ad_pad = jnp.concatenate([d1.reshape(N), jnp.zeros((TAB - N,), f32)])
    ex1, den = sc_ex(src2d, dst2d, as_pad, ad_pad)
    acc = sc_rows(h1, src2d, dst2d, ex1)
    acc0, acc1 = acc[:N], acc[ACC_R:ACC_R + N]
    den0 = den[:N].reshape(N, 1)
    den1 = den[ACC_R:ACC_R + N].reshape(N, 1)

    h2, s2, d2 = _tc2(acc0, acc1, den0, den1, h1, s1, d1, b1r,
                      W2.astype(f32), as2r, ad2r)
    as_pad2 = jnp.concatenate([s2.reshape(N), jnp.zeros((TAB - N,), f32)])
    ad_pad2 = jnp.concatenate([d2.reshape(N), jnp.zeros((TAB - N,), f32)])
    ex2, denx = sc_ex(src2d, dst2d, as_pad2, ad_pad2)
    accb = sc_rows(h2, src2d, dst2d, ex2)
    acc0b, acc1b = accb[:N], accb[ACC_R:ACC_R + N]
    den0b = denx[:N].reshape(N, 1)
    den1b = denx[ACC_R:ACC_R + N].reshape(N, 1)

    hf, s_col = _tc3a(acc0b, acc1b, den0b, den1b, h2, s2, d2, b2r,
                      aw.reshape(D, 1).astype(f32),
                      ab.reshape(1, 1).astype(f32))
    attn = _tc3b(s_col)
    batch_col = batch.astype(jnp.int32).reshape(N, 1)
    z = _tc3c(hf, attn, batch_col,
              fc1_w.astype(f32), fc1_b.reshape(1, -1).astype(f32),
              bn_g.reshape(1, -1).astype(f32), bn_b.reshape(1, -1).astype(f32),
              fc2_w.astype(f32), fc2_b.reshape(1, -1).astype(f32))
    return z


# speedup vs baseline: 37.8518x; 1.1695x over previous
import functools

import jax
import jax.numpy as jnp
from jax import lax
from jax.experimental import pallas as pl
from jax.experimental.pallas import tpu as pltpu
from jax.experimental.pallas import tpu_sc as plsc

N = 10000
E = 320000
D = 128
G = 256
NC = 2
NS = 16
LANES = 16
NW = NC * NS
CH = 128
KCH = 80
EPW = CH * KCH
EPAD = EPW * NW
ACC_R = 10240
STRIPE = ACC_R // NS
TAB = N + 16


def _leaky(x, s):
    return jnp.where(x >= 0, x, x * s)


def _tc1_body(x_ref, w_ref, asv_ref, adv_ref, h_ref, s_ref, d_ref):
    h = jnp.dot(x_ref[...], w_ref[...], preferred_element_type=jnp.float32)
    h_ref[...] = h
    s_ref[...] = jnp.sum(h * asv_ref[...], axis=1, keepdims=True)
    d_ref[...] = jnp.sum(h * adv_ref[...], axis=1, keepdims=True)


def _tc1(x, W, asv, adv):
    R = 2000
    return pl.pallas_call(
        _tc1_body,
        grid=(N // R,),
        in_specs=[
            pl.BlockSpec((R, D), lambda i: (i, 0)),
            pl.BlockSpec((D, D), lambda i: (0, 0)),
            pl.BlockSpec((1, D), lambda i: (0, 0)),
            pl.BlockSpec((1, D), lambda i: (0, 0)),
        ],
        out_specs=[
            pl.BlockSpec((R, D), lambda i: (i, 0)),
            pl.BlockSpec((R, 1), lambda i: (i, 0)),
            pl.BlockSpec((R, 1), lambda i: (i, 0)),
        ],
        out_shape=[
            jax.ShapeDtypeStruct((N, D), jnp.float32),
            jax.ShapeDtypeStruct((N, 1), jnp.float32),
            jax.ShapeDtypeStruct((N, 1), jnp.float32),
        ],
    )(x, W, asv, adv)


def _combine(acc0, acc1, den0, den1, h, s_col, d_col, b):
    e = _leaky(s_col + d_col, 0.2)
    exs = jnp.exp(e)
    num = acc0 + acc1 + exs * h
    den = den0 + den1 + exs + 1e-16
    return _leaky(num / den + b, 0.01)


def _tc2_body(acc0_ref, acc1_ref, den0_ref, den1_ref, h_ref, s_ref, d_ref,
              b_ref, w_ref, asv_ref, adv_ref, h2_ref, s2_ref, d2_ref):
    x2 = _combine(acc0_ref[...], acc1_ref[...], den0_ref[...], den1_ref[...],
                  h_ref[...], s_ref[...], d_ref[...], b_ref[...])
    h2 = jnp.dot(x2, w_ref[...], preferred_element_type=jnp.float32)
    h2_ref[...] = h2
    s2_ref[...] = jnp.sum(h2 * asv_ref[...], axis=1, keepdims=True)
    d2_ref[...] = jnp.sum(h2 * adv_ref[...], axis=1, keepdims=True)


def _tc2(acc0, acc1, den0, den1, h, s_col, d_col, b, W, asv, adv):
    R = 2000
    col = pl.BlockSpec((R, 1), lambda i: (i, 0))
    mat = pl.BlockSpec((R, D), lambda i: (i, 0))
    one = pl.BlockSpec((1, D), lambda i: (0, 0))
    return pl.pallas_call(
        _tc2_body,
        grid=(N // R,),
        in_specs=[mat, mat, col, col, mat, col, col, one,
                  pl.BlockSpec((D, D), lambda i: (0, 0)), one, one],
        out_specs=[mat, col, col],
        out_shape=[
            jax.ShapeDtypeStruct((N, D), jnp.float32),
            jax.ShapeDtypeStruct((N, 1), jnp.float32),
            jax.ShapeDtypeStruct((N, 1), jnp.float32),
        ],
    )(acc0, acc1, den0, den1, h, s_col, d_col, b, W, asv, adv)


def _tc3a_body(acc0_ref, acc1_ref, den0_ref, den1_ref, h_ref, s_ref, d_ref,
               b_ref, aw_ref, ab_ref, hf_ref, sc_ref):
    x3 = _combine(acc0_ref[...], acc1_ref[...], den0_ref[...], den1_ref[...],
                  h_ref[...], s_ref[...], d_ref[...], b_ref[...])
    hf_ref[...] = x3
    sc_ref[...] = jnp.dot(x3, aw_ref[...],
                          preferred_element_type=jnp.float32) + ab_ref[...]


def _tc3a(acc0, acc1, den0, den1, h, s_col, d_col, b, aw, ab):
    R = 2000
    col = pl.BlockSpec((R, 1), lambda i: (i, 0))
    mat = pl.BlockSpec((R, D), lambda i: (i, 0))
    return pl.pallas_call(
        _tc3a_body,
        grid=(N // R,),
        in_specs=[mat, mat, col, col, mat, col, col,
                  pl.BlockSpec((1, D), lambda i: (0, 0)),
                  pl.BlockSpec((D, 1), lambda i: (0, 0)),
                  pl.BlockSpec((1, 1), lambda i: (0, 0))],
        out_specs=[mat, col],
        out_shape=[
            jax.ShapeDtypeStruct((N, D), jnp.float32),
            jax.ShapeDtypeStruct((N, 1), jnp.float32),
        ],
    )(acc0, acc1, den0, den1, h, s_col, d_col, b, aw, ab)


def _tc3b_body(s_ref, attn_ref):
    s = s_ref[...]
    m = jnp.max(s)
    p = jnp.exp(s - m)
    attn_ref[...] = p / jnp.sum(p)


def _tc3b(s_col):
    return pl.pallas_call(
        _tc3b_body,
        out_shape=jax.ShapeDtypeStruct((N, 1), jnp.float32),
    )(s_col)


def _tc3c_body(hf_ref, attn_ref, batch_ref, f1w_ref, f1b_ref, bng_ref,
               bnb_ref, f2w_ref, f2b_ref, z_ref, g_acc):
    i = pl.program_id(0)

    @pl.when(i == 0)
    def _():
        g_acc[...] = jnp.zeros_like(g_acc)

    gid = lax.broadcasted_iota(jnp.int32, (1, G), 1)
    onehot = (batch_ref[...] == gid).astype(jnp.float32)
    w = attn_ref[...] * hf_ref[...]
    g_acc[...] += lax.dot_general(onehot, w, (((0,), (0,)), ((), ())),
                                  preferred_element_type=jnp.float32)

    @pl.when(i == pl.num_programs(0) - 1)
    def _():
        g = g_acc[...]
        z = jnp.dot(g, f1w_ref[...],
                    preferred_element_type=jnp.float32) + f1b_ref[...]
        mean = jnp.mean(z, axis=0, keepdims=True)
        var = jnp.mean((z - mean) * (z - mean), axis=0, keepdims=True)
        z = bng_ref[...] * (z - mean) / jnp.sqrt(var + 1e-5) + bnb_ref[...]
        z = _leaky(z, 0.01)
        z_ref[...] = jnp.dot(z, f2w_ref[...],
                             preferred_element_type=jnp.float32) + f2b_ref[...]


def _tc3c(hf, attn, batch_col, f1w, f1b, bng, bnb, f2w, f2b):
    R = 2000
    H = D // 2
    return pl.pallas_call(
        _tc3c_body,
        grid=(N // R,),
        in_specs=[
            pl.BlockSpec((R, D), lambda i: (i, 0)),
            pl.BlockSpec((R, 1), lambda i: (i, 0)),
            pl.BlockSpec((R, 1), lambda i: (i, 0)),
            pl.BlockSpec((D, H), lambda i: (0, 0)),
            pl.BlockSpec((1, H), lambda i: (0, 0)),
            pl.BlockSpec((1, H), lambda i: (0, 0)),
            pl.BlockSpec((1, H), lambda i: (0, 0)),
            pl.BlockSpec((H, D), lambda i: (0, 0)),
            pl.BlockSpec((1, D), lambda i: (0, 0)),
        ],
        out_specs=pl.BlockSpec((G, D), lambda i: (0, 0)),
        out_shape=jax.ShapeDtypeStruct((G, D), jnp.float32),
        scratch_shapes=[pltpu.VMEM((G, D), jnp.float32)],
    )(hf, attn, batch_col, f1w, f1b, bng, bnb, f2w, f2b)


BLK = 8
NBLK = KCH // BLK
CH2 = 64
KCH2 = EPW // CH2
SB = 32


def _make_sc_mesh():
    return plsc.VectorSubcoreMesh(core_axis_name="c", subcore_axis_name="s",
                                  num_cores=NC, num_subcores=NS)


def _make_sc_ex():
    @functools.partial(
        pl.kernel,
        out_type=[
            jax.ShapeDtypeStruct((EPAD,), jnp.float32),
            jax.ShapeDtypeStruct((NC * ACC_R,), jnp.float32),
        ],
        mesh=_make_sc_mesh(),
        compiler_params=pltpu.CompilerParams(needs_layout_passes=False),
        scratch_types=[
            pltpu.VMEM_SHARED((ACC_R,), jnp.float32),
            pltpu.VMEM((TAB,), jnp.float32),
            pltpu.VMEM((TAB,), jnp.float32),
            pltpu.VMEM((BLK, CH), jnp.int32),
            pltpu.VMEM((BLK, CH), jnp.int32),
            pltpu.VMEM((CH,), jnp.float32),
            pltpu.VMEM((STRIPE,), jnp.float32),
        ],
    )
    def sc_ex(src_hbm, dst_hbm, as_hbm, ad_hbm, ex_out, den_out,
              den_sh, as_tab, ad_tab, srcb, dstb, exb, denb):
        c = lax.axis_index("c")
        s = lax.axis_index("s")
        w = c * NS + s
        z16 = jnp.zeros((LANES,), jnp.float32)

        pltpu.sync_copy(as_hbm, as_tab)
        pltpu.sync_copy(ad_hbm, ad_tab)

        def _zden(i, _):
            denb[pl.ds(i * LANES, LANES)] = z16
            return 0
        lax.fori_loop(0, STRIPE // LANES, _zden, 0)
        pltpu.sync_copy(denb, den_sh.at[pl.ds(s * STRIPE, STRIPE)])
        plsc.subcore_barrier()

        n_real = jnp.clip(E // CH - w * KCH, 0, KCH)
        nblk = (n_real + BLK - 1) // BLK

        def _blk(b, _):
            pltpu.sync_copy(src_hbm.at[w, pl.ds(b * BLK, BLK)], srcb)
            pltpu.sync_copy(dst_hbm.at[w, pl.ds(b * BLK, BLK)], dstb)

            def _chunk(jj, _):
                def _group(g, _):
                    src16 = srcb[jj, pl.ds(g * LANES, LANES)]
                    dst16 = dstb[jj, pl.ds(g * LANES, LANES)]
                    e = (plsc.load_gather(as_tab, [src16])
                         + plsc.load_gather(ad_tab, [dst16]))
                    e = jnp.where(e >= 0, e, e * 0.2)
                    exb[pl.ds(g * LANES, LANES)] = jnp.exp(e)
                    return 0
                lax.fori_loop(0, CH // LANES, _group, 0)
                off = (w * KCH + b * BLK + jj) * CH
                pltpu.sync_copy(exb, ex_out.at[pl.ds(off, CH)])
                pltpu.sync_copy(exb, den_sh.at[dstb.at[jj]], add=True)
                return 0
            lax.fori_loop(0, jnp.minimum(BLK, n_real - b * BLK), _chunk, 0)
            return 0
        lax.fori_loop(0, nblk, _blk, 0)

        plsc.subcore_barrier()
        pltpu.sync_copy(den_sh.at[pl.ds(s * STRIPE, STRIPE)], denb)
        pltpu.sync_copy(denb, den_out.at[pl.ds(c * ACC_R + s * STRIPE,
                                               STRIPE)])

    return sc_ex


def _make_sc_rows():
    @functools.partial(
        pl.kernel,
        out_type=jax.ShapeDtypeStruct((NC * ACC_R, D), jnp.float32),
        mesh=_make_sc_mesh(),
        compiler_params=pltpu.CompilerParams(needs_layout_passes=False),
        scratch_types=[
            pltpu.VMEM_SHARED((ACC_R, D), jnp.float32),
            pltpu.VMEM((SB, CH2), jnp.int32),
            pltpu.VMEM((SB, CH2), jnp.int32),
            pltpu.VMEM((SB * CH2,), jnp.float32),
            pltpu.VMEM((CH2, D), jnp.float32),
            pltpu.VMEM((CH2, D), jnp.float32),
            pltpu.SemaphoreType.DMA,
            pltpu.SemaphoreType.DMA,
            pltpu.SemaphoreType.DMA,
            pltpu.SemaphoreType.DMA,
        ],
    )
    def sc_rows(h_hbm, src_hbm, dst_hbm, ex_hbm, acc_out,
                acc_sh, srcb, dstb, exbuf, rows0, rows1, gs0, gs1, ss0, ss1):
        c = lax.axis_index("c")
        s = lax.axis_index("s")
        w = c * NS + s
        z16 = jnp.zeros((LANES,), jnp.float32)
        bufs = ((rows0, gs0, ss0), (rows1, gs1, ss1))

        def _zrow(r, _):
            for cc in range(D // LANES):
                rows0[r, pl.ds(cc * LANES, LANES)] = z16
            return 0
        lax.fori_loop(0, CH2, _zrow, 0)
        for k in range(STRIPE // CH2):
            pltpu.sync_copy(rows0, acc_sh.at[pl.ds(s * STRIPE + k * CH2,
                                                   CH2)])
        plsc.subcore_barrier()

        n_real = jnp.clip(E // CH2 - w * KCH2, 0, KCH2)
        nsb = (n_real + SB - 1) // SB

        def _scale(rb, j):
            def _group(g, _):
                ex16 = exbuf[pl.ds(j * CH2 + g * LANES, LANES)]
                for ll in range(LANES):
                    r = g * LANES + ll
                    sv = ex16[ll]
                    for cc in range(D // LANES):
                        sl = pl.ds(cc * LANES, LANES)
                        rb[r, sl] = rb[r, sl] * sv
                return 0
            lax.fori_loop(0, CH2 // LANES, _group, 0)

        def _sb(b, _):
            base = b * SB
            m = jnp.minimum(SB, n_real - base)
            pltpu.sync_copy(src_hbm.at[w, pl.ds(base, SB)], srcb)
            pltpu.sync_copy(dst_hbm.at[w, pl.ds(base, SB)], dstb)
            pltpu.sync_copy(
                ex_hbm.at[pl.ds((w * KCH2 + base) * CH2, SB * CH2)], exbuf)

            @pl.when(m > 0)
            def _():
                pltpu.async_copy(h_hbm.at[srcb.at[0]], rows0, gs0)

            def _pair(j2, _):
                for parity in range(2):
                    rb, gs, ss = bufs[parity]
                    ro, go, so = bufs[1 - parity]
                    j = j2 * 2 + parity

                    @pl.when(j < m)
                    def _():
                        pltpu.make_async_copy(h_hbm.at[srcb.at[0]], rb,
                                              gs).wait()

                        @pl.when(j >= 1)
                        def _():
                            pltpu.make_async_copy(
                                ro, acc_sh.at[dstb.at[0]], so).wait()

                        @pl.when(j + 1 < m)
                        def _():
                            pltpu.async_copy(h_hbm.at[srcb.at[j + 1]], ro,
                                             go)

                        _scale(rb, j)
                        pltpu.async_copy(rb, acc_sh.at[dstb.at[j]], ss,
                                         add=True)
                return 0
            lax.fori_loop(0, (m + 1) // 2, _pair, 0)

            @pl.when(m > 0)
            def _():
                par = (m - 1) % 2

                @pl.when(par == 0)
                def _():
                    pltpu.make_async_copy(rows0, acc_sh.at[dstb.at[0]],
                                          ss0).wait()

                @pl.when(par == 1)
                def _():
                    pltpu.make_async_copy(rows1, acc_sh.at[dstb.at[0]],
                                          ss1).wait()
            return 0
        lax.fori_loop(0, nsb, _sb, 0)

        plsc.subcore_barrier()

        def _cp(k, _):
            off = s * STRIPE + k * CH2
            pltpu.sync_copy(acc_sh.at[pl.ds(off, CH2)], rows0)
            pltpu.sync_copy(rows0, acc_out.at[pl.ds(c * ACC_R + off, CH2)])
            return 0
        lax.fori_loop(0, STRIPE // CH2, _cp, 0)

    return sc_rows


_sc_kernels = None


def _get_sc_kernels():
    global _sc_kernels
    if _sc_kernels is None:
        _sc_kernels = (_make_sc_ex(), _make_sc_rows())
    return _sc_kernels


def kernel(x, edge_index, batch, W1, a_s1, a_d1, b1, W2, a_s2, a_d2, b2, aw,
           ab, fc1_w, fc1_b, bn_g, bn_b, fc2_w, fc2_b):
    f32 = jnp.float32
    src = edge_index[0].astype(jnp.int32)
    dst = edge_index[1].astype(jnp.int32)
    pad = EPAD - E
    src_p = jnp.concatenate([src, jnp.zeros((pad,), jnp.int32)])
    dst_p = jnp.concatenate([dst, jnp.full((pad,), N, jnp.int32)])
    src2d = src_p.reshape(NW, KCH, CH)
    dst2d = dst_p.reshape(NW, KCH, CH)
    src64 = src_p.reshape(NW, KCH2, CH2)
    dst64 = dst_p.reshape(NW, KCH2, CH2)

    as1r = a_s1.reshape(1, D).astype(f32)
    ad1r = a_d1.reshape(1, D).astype(f32)
    as2r = a_s2.reshape(1, D).astype(f32)
    ad2r = a_d2.reshape(1, D).astype(f32)
    b1r = b1.reshape(1, D).astype(f32)
    b2r = b2.reshape(1, D).astype(f32)

    sc_ex, sc_rows = _get_sc_kernels()

    h1, s1, d1 = _tc1(x.astype(f32), W1.astype(f32), as1r, ad1r)
    as_pad = jnp.concatenate([s1.reshape(N), jnp.zeros((TAB - N,), f32)])
    ad_pad = jnp.concatenate([d1.reshape(N), jnp.zeros((TAB - N,), f32)])
    ex1, den = sc_ex(src2d, dst2d, as_pad, ad_pad)
    acc = sc_rows(h1, src64, dst64, ex1)
    acc0, acc1 = acc[:N], acc[ACC_R:ACC_R + N]
    den0 = den[:N].reshape(N, 1)
    den1 = den[ACC_R:ACC_R + N].reshape(N, 1)

    h2, s2, d2 = _tc2(acc0, acc1, den0, den1, h1, s1, d1, b1r,
                      W2.astype(f32), as2r, ad2r)
    as_pad2 = jnp.concatenate([s2.reshape(N), jnp.zeros((TAB - N,), f32)])
    ad_pad2 = jnp.concatenate([d2.reshape(N), jnp.zeros((TAB - N,), f32)])
    ex2, denx = sc_ex(src2d, dst2d, as_pad2, ad_pad2)
    accb = sc_rows(h2, src64, dst64, ex2)
    acc0b, acc1b = accb[:N], accb[ACC_R:ACC_R + N]
    den0b = denx[:N].reshape(N, 1)
    den1b = denx[ACC_R:ACC_R + N].reshape(N, 1)

    hf, s_col = _tc3a(acc0b, acc1b, den0b, den1b, h2, s2, d2, b2r,
                      aw.reshape(D, 1).astype(f32),
                      ab.reshape(1, 1).astype(f32))
    attn = _tc3b(s_col)
    batch_col = batch.astype(jnp.int32).reshape(N, 1)
    z = _tc3c(hf, attn, batch_col,
              fc1_w.astype(f32), fc1_b.reshape(1, -1).astype(f32),
              bn_g.reshape(1, -1).astype(f32), bn_b.reshape(1, -1).astype(f32),
              fc2_w.astype(f32), fc2_b.reshape(1, -1).astype(f32))
    return z


# speedup vs baseline: 39.3078x; 1.0385x over previous
import functools

import jax
import jax.numpy as jnp
from jax import lax
from jax.experimental import pallas as pl
from jax.experimental.pallas import tpu as pltpu
from jax.experimental.pallas import tpu_sc as plsc

N = 10000
E = 320000
D = 128
G = 256
NC = 2
NS = 16
LANES = 16
NW = NC * NS
CH = 128
KCH = 80
EPW = CH * KCH
EPAD = EPW * NW
ACC_R = 10240
STRIPE = ACC_R // NS
TAB = N + 16


def _leaky(x, s):
    return jnp.where(x >= 0, x, x * s)


def _tc1_body(x_ref, w_ref, asv_ref, adv_ref, h_ref, s_ref, d_ref):
    h = jnp.dot(x_ref[...], w_ref[...], preferred_element_type=jnp.float32)
    h_ref[...] = h
    s_ref[...] = jnp.sum(h * asv_ref[...], axis=1, keepdims=True)
    d_ref[...] = jnp.sum(h * adv_ref[...], axis=1, keepdims=True)


def _tc1(x, W, asv, adv):
    R = 2000
    return pl.pallas_call(
        _tc1_body,
        grid=(N // R,),
        in_specs=[
            pl.BlockSpec((R, D), lambda i: (i, 0)),
            pl.BlockSpec((D, D), lambda i: (0, 0)),
            pl.BlockSpec((1, D), lambda i: (0, 0)),
            pl.BlockSpec((1, D), lambda i: (0, 0)),
        ],
        out_specs=[
            pl.BlockSpec((R, D), lambda i: (i, 0)),
            pl.BlockSpec((R, 1), lambda i: (i, 0)),
            pl.BlockSpec((R, 1), lambda i: (i, 0)),
        ],
        out_shape=[
            jax.ShapeDtypeStruct((N, D), jnp.float32),
            jax.ShapeDtypeStruct((N, 1), jnp.float32),
            jax.ShapeDtypeStruct((N, 1), jnp.float32),
        ],
    )(x, W, asv, adv)


def _combine(acc0, acc1, den0, den1, h, s_col, d_col, b):
    e = _leaky(s_col + d_col, 0.2)
    exs = jnp.exp(e)
    num = acc0 + acc1 + exs * h
    den = den0 + den1 + exs + 1e-16
    return _leaky(num / den + b, 0.01)


def _tc2_body(acc0_ref, acc1_ref, den0_ref, den1_ref, h_ref, s_ref, d_ref,
              b_ref, w_ref, asv_ref, adv_ref, h2_ref, s2_ref, d2_ref):
    x2 = _combine(acc0_ref[...], acc1_ref[...], den0_ref[...], den1_ref[...],
                  h_ref[...], s_ref[...], d_ref[...], b_ref[...])
    h2 = jnp.dot(x2, w_ref[...], preferred_element_type=jnp.float32)
    h2_ref[...] = h2
    s2_ref[...] = jnp.sum(h2 * asv_ref[...], axis=1, keepdims=True)
    d2_ref[...] = jnp.sum(h2 * adv_ref[...], axis=1, keepdims=True)


def _tc2(acc0, acc1, den0, den1, h, s_col, d_col, b, W, asv, adv):
    R = 2000
    col = pl.BlockSpec((R, 1), lambda i: (i, 0))
    mat = pl.BlockSpec((R, D), lambda i: (i, 0))
    one = pl.BlockSpec((1, D), lambda i: (0, 0))
    return pl.pallas_call(
        _tc2_body,
        grid=(N // R,),
        in_specs=[mat, mat, col, col, mat, col, col, one,
                  pl.BlockSpec((D, D), lambda i: (0, 0)), one, one],
        out_specs=[mat, col, col],
        out_shape=[
            jax.ShapeDtypeStruct((N, D), jnp.float32),
            jax.ShapeDtypeStruct((N, 1), jnp.float32),
            jax.ShapeDtypeStruct((N, 1), jnp.float32),
        ],
    )(acc0, acc1, den0, den1, h, s_col, d_col, b, W, asv, adv)


def _tc3a_body(acc0_ref, acc1_ref, den0_ref, den1_ref, h_ref, s_ref, d_ref,
               b_ref, aw_ref, ab_ref, hf_ref, sc_ref):
    x3 = _combine(acc0_ref[...], acc1_ref[...], den0_ref[...], den1_ref[...],
                  h_ref[...], s_ref[...], d_ref[...], b_ref[...])
    hf_ref[...] = x3
    sc_ref[...] = jnp.dot(x3, aw_ref[...],
                          preferred_element_type=jnp.float32) + ab_ref[...]


def _tc3a(acc0, acc1, den0, den1, h, s_col, d_col, b, aw, ab):
    R = 2000
    col = pl.BlockSpec((R, 1), lambda i: (i, 0))
    mat = pl.BlockSpec((R, D), lambda i: (i, 0))
    return pl.pallas_call(
        _tc3a_body,
        grid=(N // R,),
        in_specs=[mat, mat, col, col, mat, col, col,
                  pl.BlockSpec((1, D), lambda i: (0, 0)),
                  pl.BlockSpec((D, 1), lambda i: (0, 0)),
                  pl.BlockSpec((1, 1), lambda i: (0, 0))],
        out_specs=[mat, col],
        out_shape=[
            jax.ShapeDtypeStruct((N, D), jnp.float32),
            jax.ShapeDtypeStruct((N, 1), jnp.float32),
        ],
    )(acc0, acc1, den0, den1, h, s_col, d_col, b, aw, ab)


def _tc3b_body(s_ref, attn_ref):
    s = s_ref[...]
    m = jnp.max(s)
    p = jnp.exp(s - m)
    attn_ref[...] = p / jnp.sum(p)


def _tc3b(s_col):
    return pl.pallas_call(
        _tc3b_body,
        out_shape=jax.ShapeDtypeStruct((N, 1), jnp.float32),
    )(s_col)


def _tc3c_body(hf_ref, attn_ref, batch_ref, f1w_ref, f1b_ref, bng_ref,
               bnb_ref, f2w_ref, f2b_ref, z_ref, g_acc):
    i = pl.program_id(0)

    @pl.when(i == 0)
    def _():
        g_acc[...] = jnp.zeros_like(g_acc)

    gid = lax.broadcasted_iota(jnp.int32, (1, G), 1)
    onehot = (batch_ref[...] == gid).astype(jnp.float32)
    w = attn_ref[...] * hf_ref[...]
    g_acc[...] += lax.dot_general(onehot, w, (((0,), (0,)), ((), ())),
                                  preferred_element_type=jnp.float32)

    @pl.when(i == pl.num_programs(0) - 1)
    def _():
        g = g_acc[...]
        z = jnp.dot(g, f1w_ref[...],
                    preferred_element_type=jnp.float32) + f1b_ref[...]
        mean = jnp.mean(z, axis=0, keepdims=True)
        var = jnp.mean((z - mean) * (z - mean), axis=0, keepdims=True)
        z = bng_ref[...] * (z - mean) / jnp.sqrt(var + 1e-5) + bnb_ref[...]
        z = _leaky(z, 0.01)
        z_ref[...] = jnp.dot(z, f2w_ref[...],
                             preferred_element_type=jnp.float32) + f2b_ref[...]


def _tc3c(hf, attn, batch_col, f1w, f1b, bng, bnb, f2w, f2b):
    R = 2000
    H = D // 2
    return pl.pallas_call(
        _tc3c_body,
        grid=(N // R,),
        in_specs=[
            pl.BlockSpec((R, D), lambda i: (i, 0)),
            pl.BlockSpec((R, 1), lambda i: (i, 0)),
            pl.BlockSpec((R, 1), lambda i: (i, 0)),
            pl.BlockSpec((D, H), lambda i: (0, 0)),
            pl.BlockSpec((1, H), lambda i: (0, 0)),
            pl.BlockSpec((1, H), lambda i: (0, 0)),
            pl.BlockSpec((1, H), lambda i: (0, 0)),
            pl.BlockSpec((H, D), lambda i: (0, 0)),
            pl.BlockSpec((1, D), lambda i: (0, 0)),
        ],
        out_specs=pl.BlockSpec((G, D), lambda i: (0, 0)),
        out_shape=jax.ShapeDtypeStruct((G, D), jnp.float32),
        scratch_shapes=[pltpu.VMEM((G, D), jnp.float32)],
    )(hf, attn, batch_col, f1w, f1b, bng, bnb, f2w, f2b)


BLK = 8
NBLK = KCH // BLK
CH2 = 64
KCH2 = EPW // CH2
SB = 32
SBX = 32
EXR = 96


def _make_sc_mesh():
    return plsc.VectorSubcoreMesh(core_axis_name="c", subcore_axis_name="s",
                                  num_cores=NC, num_subcores=NS)


def _make_sc_ex():
    @functools.partial(
        pl.kernel,
        out_type=[
            jax.ShapeDtypeStruct((NW * EXR, CH), jnp.float32),
            jax.ShapeDtypeStruct((NC * ACC_R,), jnp.float32),
        ],
        mesh=_make_sc_mesh(),
        compiler_params=pltpu.CompilerParams(needs_layout_passes=False),
        scratch_types=[
            pltpu.VMEM_SHARED((ACC_R,), jnp.float32),
            pltpu.VMEM((TAB,), jnp.float32),
            pltpu.VMEM((TAB,), jnp.float32),
            pltpu.VMEM((SBX, CH), jnp.int32),
            pltpu.VMEM((SBX, CH), jnp.int32),
            pltpu.VMEM((SBX, CH), jnp.float32),
            pltpu.VMEM((STRIPE,), jnp.float32),
            pltpu.SemaphoreType.DMA,
        ],
    )
    def sc_ex(src_hbm, dst_hbm, as_hbm, ad_hbm, ex_out, den_out,
              den_sh, as_tab, ad_tab, srcb, dstb, exsb, denb, ssem):
        c = lax.axis_index("c")
        s = lax.axis_index("s")
        w = c * NS + s
        z16 = jnp.zeros((LANES,), jnp.float32)

        pltpu.sync_copy(as_hbm, as_tab)
        pltpu.sync_copy(ad_hbm, ad_tab)

        def _zden(i, _):
            denb[pl.ds(i * LANES, LANES)] = z16
            return 0
        lax.fori_loop(0, STRIPE // LANES, _zden, 0)
        pltpu.sync_copy(denb, den_sh.at[pl.ds(s * STRIPE, STRIPE)])
        plsc.subcore_barrier()

        n_real = jnp.clip(E // CH - w * KCH, 0, KCH)
        nsb = (n_real + SBX - 1) // SBX

        def _sb(b, _):
            base = b * SBX
            m = jnp.minimum(SBX, n_real - base)
            pltpu.sync_copy(src_hbm.at[w, pl.ds(base, SBX)], srcb)
            pltpu.sync_copy(dst_hbm.at[w, pl.ds(base, SBX)], dstb)

            def _chunk(jj, _):
                def _group(g, _):
                    src16 = srcb[jj, pl.ds(g * LANES, LANES)]
                    dst16 = dstb[jj, pl.ds(g * LANES, LANES)]
                    e = (plsc.load_gather(as_tab, [src16])
                         + plsc.load_gather(ad_tab, [dst16]))
                    e = jnp.where(e >= 0, e, e * 0.2)
                    exsb[jj, pl.ds(g * LANES, LANES)] = jnp.exp(e)
                    return 0
                lax.fori_loop(0, CH // LANES, _group, 0)
                pltpu.sync_copy(exsb.at[jj], den_sh.at[dstb.at[jj]],
                                add=True)
                return 0
            lax.fori_loop(0, m, _chunk, 0)

            pltpu.sync_copy(
                exsb, ex_out.at[pl.ds(pl.multiple_of(w * EXR + base, 8),
                                      SBX)])
            return 0
        lax.fori_loop(0, nsb, _sb, 0)

        plsc.subcore_barrier()
        pltpu.sync_copy(den_sh.at[pl.ds(s * STRIPE, STRIPE)], denb)
        pltpu.sync_copy(denb, den_out.at[pl.ds(c * ACC_R + s * STRIPE,
                                               STRIPE)])

    return sc_ex


def _make_sc_rows():
    @functools.partial(
        pl.kernel,
        out_type=jax.ShapeDtypeStruct((NC * ACC_R, D), jnp.float32),
        mesh=_make_sc_mesh(),
        compiler_params=pltpu.CompilerParams(needs_layout_passes=False),
        scratch_types=[
            pltpu.VMEM_SHARED((ACC_R, D), jnp.float32),
            pltpu.VMEM((SB, CH2), jnp.int32),
            pltpu.VMEM((SB, CH2), jnp.int32),
            pltpu.VMEM((SB * CH2 // CH, CH), jnp.float32),
            pltpu.VMEM((CH2, D), jnp.float32),
            pltpu.VMEM((CH2, D), jnp.float32),
            pltpu.SemaphoreType.DMA,
            pltpu.SemaphoreType.DMA,
            pltpu.SemaphoreType.DMA,
            pltpu.SemaphoreType.DMA,
        ],
    )
    def sc_rows(h_hbm, src_hbm, dst_hbm, ex_hbm, acc_out,
                acc_sh, srcb, dstb, exbuf, rows0, rows1, gs0, gs1, ss0, ss1):
        c = lax.axis_index("c")
        s = lax.axis_index("s")
        w = c * NS + s
        z16 = jnp.zeros((LANES,), jnp.float32)
        bufs = ((rows0, gs0, ss0), (rows1, gs1, ss1))

        def _zrow(r, _):
            for cc in range(D // LANES):
                rows0[r, pl.ds(cc * LANES, LANES)] = z16
            return 0
        lax.fori_loop(0, CH2, _zrow, 0)
        for k in range(STRIPE // CH2):
            pltpu.sync_copy(rows0, acc_sh.at[pl.ds(s * STRIPE + k * CH2,
                                                   CH2)])
        plsc.subcore_barrier()

        n_real = jnp.clip(E // CH2 - w * KCH2, 0, KCH2)
        nsb = (n_real + SB - 1) // SB

        def _scale(rb, j):
            jr = j // 2
            jc = (j % 2) * CH2

            def _group(g, _):
                ex16 = exbuf[jr, pl.ds(jc + g * LANES, LANES)]
                for ll in range(LANES):
                    r = g * LANES + ll
                    sv = ex16[ll]
                    for cc in range(D // LANES):
                        sl = pl.ds(cc * LANES, LANES)
                        rb[r, sl] = rb[r, sl] * sv
                return 0
            lax.fori_loop(0, CH2 // LANES, _group, 0)

        def _sb(b, _):
            base = b * SB
            m = jnp.minimum(SB, n_real - base)
            pltpu.sync_copy(src_hbm.at[w, pl.ds(base, SB)], srcb)
            pltpu.sync_copy(dst_hbm.at[w, pl.ds(base, SB)], dstb)
            pltpu.sync_copy(
                ex_hbm.at[pl.ds(pl.multiple_of(w * EXR + base * CH2 // CH, 8),
                                SB * CH2 // CH)],
                exbuf)

            @pl.when(m > 0)
            def _():
                pltpu.async_copy(h_hbm.at[srcb.at[0]], rows0, gs0)

            def _pair(j2, _):
                for parity in range(2):
                    rb, gs, ss = bufs[parity]
                    ro, go, so = bufs[1 - parity]
                    j = j2 * 2 + parity

                    @pl.when(j < m)
                    def _():
                        pltpu.make_async_copy(h_hbm.at[srcb.at[0]], rb,
                                              gs).wait()

                        @pl.when(j >= 1)
                        def _():
                            pltpu.make_async_copy(
                                ro, acc_sh.at[dstb.at[0]], so).wait()

                        @pl.when(j + 1 < m)
                        def _():
                            pltpu.async_copy(h_hbm.at[srcb.at[j + 1]], ro,
                                             go)

                        _scale(rb, j)
                        pltpu.async_copy(rb, acc_sh.at[dstb.at[j]], ss,
                                         add=True)
                return 0
            lax.fori_loop(0, (m + 1) // 2, _pair, 0)

            @pl.when(m > 0)
            def _():
                par = (m - 1) % 2

                @pl.when(par == 0)
                def _():
                    pltpu.make_async_copy(rows0, acc_sh.at[dstb.at[0]],
                                          ss0).wait()

                @pl.when(par == 1)
                def _():
                    pltpu.make_async_copy(rows1, acc_sh.at[dstb.at[0]],
                                          ss1).wait()
            return 0
        lax.fori_loop(0, nsb, _sb, 0)

        plsc.subcore_barrier()

        def _cp(k, _):
            off = s * STRIPE + k * CH2
            pltpu.sync_copy(acc_sh.at[pl.ds(off, CH2)], rows0)
            pltpu.sync_copy(rows0, acc_out.at[pl.ds(c * ACC_R + off, CH2)])
            return 0
        lax.fori_loop(0, STRIPE // CH2, _cp, 0)

    return sc_rows


_sc_kernels = None


def _get_sc_kernels():
    global _sc_kernels
    if _sc_kernels is None:
        _sc_kernels = (_make_sc_ex(), _make_sc_rows())
    return _sc_kernels


def kernel(x, edge_index, batch, W1, a_s1, a_d1, b1, W2, a_s2, a_d2, b2, aw,
           ab, fc1_w, fc1_b, bn_g, bn_b, fc2_w, fc2_b):
    f32 = jnp.float32
    src = edge_index[0].astype(jnp.int32)
    dst = edge_index[1].astype(jnp.int32)
    pad = EPAD - E
    src_p = jnp.concatenate([src, jnp.zeros((pad,), jnp.int32)])
    dst_p = jnp.concatenate([dst, jnp.full((pad,), N, jnp.int32)])
    src2d = src_p.reshape(NW, KCH, CH)
    dst2d = dst_p.reshape(NW, KCH, CH)
    src64 = src_p.reshape(NW, KCH2, CH2)
    dst64 = dst_p.reshape(NW, KCH2, CH2)

    as1r = a_s1.reshape(1, D).astype(f32)
    ad1r = a_d1.reshape(1, D).astype(f32)
    as2r = a_s2.reshape(1, D).astype(f32)
    ad2r = a_d2.reshape(1, D).astype(f32)
    b1r = b1.reshape(1, D).astype(f32)
    b2r = b2.reshape(1, D).astype(f32)

    sc_ex, sc_rows = _get_sc_kernels()

    h1, s1, d1 = _tc1(x.astype(f32), W1.astype(f32), as1r, ad1r)
    as_pad = jnp.concatenate([s1.reshape(N), jnp.zeros((TAB - N,), f32)])
    ad_pad = jnp.concatenate([d1.reshape(N), jnp.zeros((TAB - N,), f32)])
    ex1, den = sc_ex(src2d, dst2d, as_pad, ad_pad)
    acc = sc_rows(h1, src64, dst64, ex1)
    acc0, acc1 = acc[:N], acc[ACC_R:ACC_R + N]
    den0 = den[:N].reshape(N, 1)
    den1 = den[ACC_R:ACC_R + N].reshape(N, 1)

    h2, s2, d2 = _tc2(acc0, acc1, den0, den1, h1, s1, d1, b1r,
                      W2.astype(f32), as2r, ad2r)
    as_pad2 = jnp.concatenate([s2.reshape(N), jnp.zeros((TAB - N,), f32)])
    ad_pad2 = jnp.concatenate([d2.reshape(N), jnp.zeros((TAB - N,), f32)])
    ex2, denx = sc_ex(src2d, dst2d, as_pad2, ad_pad2)
    accb = sc_rows(h2, src64, dst64, ex2)
    acc0b, acc1b = accb[:N], accb[ACC_R:ACC_R + N]
    den0b = denx[:N].reshape(N, 1)
    den1b = denx[ACC_R:ACC_R + N].reshape(N, 1)

    hf, s_col = _tc3a(acc0b, acc1b, den0b, den1b, h2, s2, d2, b2r,
                      aw.reshape(D, 1).astype(f32),
                      ab.reshape(1, 1).astype(f32))
    attn = _tc3b(s_col)
    batch_col = batch.astype(jnp.int32).reshape(N, 1)
    z = _tc3c(hf, attn, batch_col,
              fc1_w.astype(f32), fc1_b.reshape(1, -1).astype(f32),
              bn_g.reshape(1, -1).astype(f32), bn_b.reshape(1, -1).astype(f32),
              fc2_w.astype(f32), fc2_b.reshape(1, -1).astype(f32))
    return z


# speedup vs baseline: 40.3711x; 1.0271x over previous
import functools

import jax
import jax.numpy as jnp
from jax import lax
from jax.experimental import pallas as pl
from jax.experimental.pallas import tpu as pltpu
from jax.experimental.pallas import tpu_sc as plsc

N = 10000
E = 320000
D = 128
G = 256
NC = 2
NS = 16
LANES = 16
NW = NC * NS
CH = 128
KCH = 80
EPW = CH * KCH
EPAD = EPW * NW
ACC_R = 10240
STRIPE = ACC_R // NS
TAB = N + 16


def _leaky(x, s):
    return jnp.where(x >= 0, x, x * s)


def _tc1_body(x_ref, w_ref, asv_ref, adv_ref, h_ref, s_ref, d_ref):
    h = jnp.dot(x_ref[...], w_ref[...], preferred_element_type=jnp.float32)
    h_ref[...] = h
    s_ref[...] = jnp.sum(h * asv_ref[...], axis=1, keepdims=True)
    d_ref[...] = jnp.sum(h * adv_ref[...], axis=1, keepdims=True)


def _tc1(x, W, asv, adv):
    R = 2000
    return pl.pallas_call(
        _tc1_body,
        grid=(N // R,),
        in_specs=[
            pl.BlockSpec((R, D), lambda i: (i, 0)),
            pl.BlockSpec((D, D), lambda i: (0, 0)),
            pl.BlockSpec((1, D), lambda i: (0, 0)),
            pl.BlockSpec((1, D), lambda i: (0, 0)),
        ],
        out_specs=[
            pl.BlockSpec((R, D), lambda i: (i, 0)),
            pl.BlockSpec((R, 1), lambda i: (i, 0)),
            pl.BlockSpec((R, 1), lambda i: (i, 0)),
        ],
        out_shape=[
            jax.ShapeDtypeStruct((N, D), jnp.float32),
            jax.ShapeDtypeStruct((N, 1), jnp.float32),
            jax.ShapeDtypeStruct((N, 1), jnp.float32),
        ],
    )(x, W, asv, adv)


def _combine(acc0, acc1, den0, den1, h, s_col, d_col, b):
    e = _leaky(s_col + d_col, 0.2)
    exs = jnp.exp(e)
    num = acc0 + acc1 + exs * h
    den = den0 + den1 + exs + 1e-16
    return _leaky(num / den + b, 0.01)


def _tc2_body(acc0_ref, acc1_ref, den0_ref, den1_ref, h_ref, s_ref, d_ref,
              b_ref, w_ref, asv_ref, adv_ref, h2_ref, s2_ref, d2_ref):
    x2 = _combine(acc0_ref[...], acc1_ref[...], den0_ref[...], den1_ref[...],
                  h_ref[...], s_ref[...], d_ref[...], b_ref[...])
    h2 = jnp.dot(x2, w_ref[...], preferred_element_type=jnp.float32)
    h2_ref[...] = h2
    s2_ref[...] = jnp.sum(h2 * asv_ref[...], axis=1, keepdims=True)
    d2_ref[...] = jnp.sum(h2 * adv_ref[...], axis=1, keepdims=True)


def _tc2(acc0, acc1, den0, den1, h, s_col, d_col, b, W, asv, adv):
    R = 2000
    col = pl.BlockSpec((R, 1), lambda i: (i, 0))
    mat = pl.BlockSpec((R, D), lambda i: (i, 0))
    one = pl.BlockSpec((1, D), lambda i: (0, 0))
    return pl.pallas_call(
        _tc2_body,
        grid=(N // R,),
        in_specs=[mat, mat, col, col, mat, col, col, one,
                  pl.BlockSpec((D, D), lambda i: (0, 0)), one, one],
        out_specs=[mat, col, col],
        out_shape=[
            jax.ShapeDtypeStruct((N, D), jnp.float32),
            jax.ShapeDtypeStruct((N, 1), jnp.float32),
            jax.ShapeDtypeStruct((N, 1), jnp.float32),
        ],
    )(acc0, acc1, den0, den1, h, s_col, d_col, b, W, asv, adv)


def _tc3a_body(acc0_ref, acc1_ref, den0_ref, den1_ref, h_ref, s_ref, d_ref,
               b_ref, aw_ref, ab_ref, hf_ref, sc_ref):
    x3 = _combine(acc0_ref[...], acc1_ref[...], den0_ref[...], den1_ref[...],
                  h_ref[...], s_ref[...], d_ref[...], b_ref[...])
    hf_ref[...] = x3
    sc_ref[...] = jnp.dot(x3, aw_ref[...],
                          preferred_element_type=jnp.float32) + ab_ref[...]


def _tc3a(acc0, acc1, den0, den1, h, s_col, d_col, b, aw, ab):
    R = 2000
    col = pl.BlockSpec((R, 1), lambda i: (i, 0))
    mat = pl.BlockSpec((R, D), lambda i: (i, 0))
    return pl.pallas_call(
        _tc3a_body,
        grid=(N // R,),
        in_specs=[mat, mat, col, col, mat, col, col,
                  pl.BlockSpec((1, D), lambda i: (0, 0)),
                  pl.BlockSpec((D, 1), lambda i: (0, 0)),
                  pl.BlockSpec((1, 1), lambda i: (0, 0))],
        out_specs=[mat, col],
        out_shape=[
            jax.ShapeDtypeStruct((N, D), jnp.float32),
            jax.ShapeDtypeStruct((N, 1), jnp.float32),
        ],
    )(acc0, acc1, den0, den1, h, s_col, d_col, b, aw, ab)


def _tc3b_body(s_ref, attn_ref):
    s = s_ref[...]
    m = jnp.max(s)
    p = jnp.exp(s - m)
    attn_ref[...] = p / jnp.sum(p)


def _tc3b(s_col):
    return pl.pallas_call(
        _tc3b_body,
        out_shape=jax.ShapeDtypeStruct((N, 1), jnp.float32),
    )(s_col)


def _tc3c_body(hf_ref, attn_ref, batch_ref, f1w_ref, f1b_ref, bng_ref,
               bnb_ref, f2w_ref, f2b_ref, z_ref, g_acc):
    i = pl.program_id(0)

    @pl.when(i == 0)
    def _():
        g_acc[...] = jnp.zeros_like(g_acc)

    gid = lax.broadcasted_iota(jnp.int32, (1, G), 1)
    onehot = (batch_ref[...] == gid).astype(jnp.float32)
    w = attn_ref[...] * hf_ref[...]
    g_acc[...] += lax.dot_general(onehot, w, (((0,), (0,)), ((), ())),
                                  preferred_element_type=jnp.float32)

    @pl.when(i == pl.num_programs(0) - 1)
    def _():
        g = g_acc[...]
        z = jnp.dot(g, f1w_ref[...],
                    preferred_element_type=jnp.float32) + f1b_ref[...]
        mean = jnp.mean(z, axis=0, keepdims=True)
        var = jnp.mean((z - mean) * (z - mean), axis=0, keepdims=True)
        z = bng_ref[...] * (z - mean) / jnp.sqrt(var + 1e-5) + bnb_ref[...]
        z = _leaky(z, 0.01)
        z_ref[...] = jnp.dot(z, f2w_ref[...],
                             preferred_element_type=jnp.float32) + f2b_ref[...]


def _tc3c(hf, attn, batch_col, f1w, f1b, bng, bnb, f2w, f2b):
    R = 2000
    H = D // 2
    return pl.pallas_call(
        _tc3c_body,
        grid=(N // R,),
        in_specs=[
            pl.BlockSpec((R, D), lambda i: (i, 0)),
            pl.BlockSpec((R, 1), lambda i: (i, 0)),
            pl.BlockSpec((R, 1), lambda i: (i, 0)),
            pl.BlockSpec((D, H), lambda i: (0, 0)),
            pl.BlockSpec((1, H), lambda i: (0, 0)),
            pl.BlockSpec((1, H), lambda i: (0, 0)),
            pl.BlockSpec((1, H), lambda i: (0, 0)),
            pl.BlockSpec((H, D), lambda i: (0, 0)),
            pl.BlockSpec((1, D), lambda i: (0, 0)),
        ],
        out_specs=pl.BlockSpec((G, D), lambda i: (0, 0)),
        out_shape=jax.ShapeDtypeStruct((G, D), jnp.float32),
        scratch_shapes=[pltpu.VMEM((G, D), jnp.float32)],
    )(hf, attn, batch_col, f1w, f1b, bng, bnb, f2w, f2b)


BLK = 8
NBLK = KCH // BLK
CH2 = 64
KCH2 = EPW // CH2
SB = 32
SBX = 32
EXR = 96


def _make_sc_mesh():
    return plsc.VectorSubcoreMesh(core_axis_name="c", subcore_axis_name="s",
                                  num_cores=NC, num_subcores=NS)


def _make_sc_ex():
    @functools.partial(
        pl.kernel,
        out_type=[
            jax.ShapeDtypeStruct((NW * EXR, CH), jnp.float32),
            jax.ShapeDtypeStruct((NC * ACC_R,), jnp.float32),
        ],
        mesh=_make_sc_mesh(),
        compiler_params=pltpu.CompilerParams(needs_layout_passes=False),
        scratch_types=[
            pltpu.VMEM_SHARED((ACC_R,), jnp.float32),
            pltpu.VMEM((TAB,), jnp.float32),
            pltpu.VMEM((TAB,), jnp.float32),
            pltpu.VMEM((SBX, CH), jnp.int32),
            pltpu.VMEM((SBX, CH), jnp.int32),
            pltpu.VMEM((SBX, CH), jnp.float32),
            pltpu.VMEM((STRIPE,), jnp.float32),
            pltpu.SemaphoreType.DMA,
        ],
    )
    def sc_ex(src_hbm, dst_hbm, as_hbm, ad_hbm, ex_out, den_out,
              den_sh, as_tab, ad_tab, srcb, dstb, exsb, denb, ssem):
        c = lax.axis_index("c")
        s = lax.axis_index("s")
        w = c * NS + s
        z16 = jnp.zeros((LANES,), jnp.float32)

        pltpu.sync_copy(as_hbm, as_tab)
        pltpu.sync_copy(ad_hbm, ad_tab)

        def _zden(i, _):
            denb[pl.ds(i * LANES, LANES)] = z16
            return 0
        lax.fori_loop(0, STRIPE // LANES, _zden, 0)
        pltpu.sync_copy(denb, den_sh.at[pl.ds(s * STRIPE, STRIPE)])
        plsc.subcore_barrier()

        n_real = jnp.clip(E // CH - w * KCH, 0, KCH)
        nsb = (n_real + SBX - 1) // SBX

        def _sb(b, _):
            base = b * SBX
            m = jnp.minimum(SBX, n_real - base)
            pltpu.sync_copy(src_hbm.at[w, pl.ds(base, SBX)], srcb)
            pltpu.sync_copy(dst_hbm.at[w, pl.ds(base, SBX)], dstb)

            def _chunk(jj, _):
                def _group(g, _):
                    src16 = srcb[jj, pl.ds(g * LANES, LANES)]
                    dst16 = dstb[jj, pl.ds(g * LANES, LANES)]
                    e = (plsc.load_gather(as_tab, [src16])
                         + plsc.load_gather(ad_tab, [dst16]))
                    e = jnp.where(e >= 0, e, e * 0.2)
                    exsb[jj, pl.ds(g * LANES, LANES)] = jnp.exp(e)
                    return 0
                lax.fori_loop(0, CH // LANES, _group, 0)
                pltpu.async_copy(exsb.at[jj], den_sh.at[dstb.at[jj]], ssem,
                                 add=True)
                return 0
            lax.fori_loop(0, m, _chunk, 0)

            def _drain(jj, _):
                pltpu.make_async_copy(exsb.at[0], den_sh.at[dstb.at[0]],
                                      ssem).wait()
                return 0
            lax.fori_loop(0, m, _drain, 0)

            pltpu.sync_copy(
                exsb, ex_out.at[pl.ds(pl.multiple_of(w * EXR + base, 8),
                                      SBX)])
            return 0
        lax.fori_loop(0, nsb, _sb, 0)

        plsc.subcore_barrier()
        pltpu.sync_copy(den_sh.at[pl.ds(s * STRIPE, STRIPE)], denb)
        pltpu.sync_copy(denb, den_out.at[pl.ds(c * ACC_R + s * STRIPE,
                                               STRIPE)])

    return sc_ex


def _make_sc_rows():
    @functools.partial(
        pl.kernel,
        out_type=jax.ShapeDtypeStruct((NC * ACC_R, D), jnp.float32),
        mesh=_make_sc_mesh(),
        compiler_params=pltpu.CompilerParams(needs_layout_passes=False),
        scratch_types=[
            pltpu.VMEM_SHARED((ACC_R, D), jnp.float32),
            pltpu.VMEM((SB, CH2), jnp.int32),
            pltpu.VMEM((SB, CH2), jnp.int32),
            pltpu.VMEM((SB * CH2 // CH, CH), jnp.float32),
            pltpu.VMEM((CH2, D), jnp.float32),
            pltpu.VMEM((CH2, D), jnp.float32),
            pltpu.SemaphoreType.DMA,
            pltpu.SemaphoreType.DMA,
            pltpu.SemaphoreType.DMA,
            pltpu.SemaphoreType.DMA,
        ],
    )
    def sc_rows(h_hbm, src_hbm, dst_hbm, ex_hbm, acc_out,
                acc_sh, srcb, dstb, exbuf, rows0, rows1, gs0, gs1, ss0, ss1):
        c = lax.axis_index("c")
        s = lax.axis_index("s")
        w = c * NS + s
        z16 = jnp.zeros((LANES,), jnp.float32)
        bufs = ((rows0, gs0, ss0), (rows1, gs1, ss1))

        def _zrow(r, _):
            for cc in range(D // LANES):
                rows0[r, pl.ds(cc * LANES, LANES)] = z16
            return 0
        lax.fori_loop(0, CH2, _zrow, 0)
        for k in range(STRIPE // CH2):
            pltpu.sync_copy(rows0, acc_sh.at[pl.ds(s * STRIPE + k * CH2,
                                                   CH2)])
        plsc.subcore_barrier()

        n_real = jnp.clip(E // CH2 - w * KCH2, 0, KCH2)
        nsb = (n_real + SB - 1) // SB

        def _scale(rb, j):
            jr = j // 2
            jc = (j % 2) * CH2

            def _group(g, _):
                ex16 = exbuf[jr, pl.ds(jc + g * LANES, LANES)]
                for ll in range(LANES):
                    r = g * LANES + ll
                    sv = ex16[ll]
                    for cc in range(D // LANES):
                        sl = pl.ds(cc * LANES, LANES)
                        rb[r, sl] = rb[r, sl] * sv
                return 0
            lax.fori_loop(0, CH2 // LANES, _group, 0)

        def _sb(b, _):
            base = b * SB
            m = jnp.minimum(SB, n_real - base)
            pltpu.sync_copy(src_hbm.at[w, pl.ds(base, SB)], srcb)
            pltpu.sync_copy(dst_hbm.at[w, pl.ds(base, SB)], dstb)
            pltpu.sync_copy(
                ex_hbm.at[pl.ds(pl.multiple_of(w * EXR + base * CH2 // CH, 8),
                                SB * CH2 // CH)],
                exbuf)

            @pl.when(m > 0)
            def _():
                pltpu.async_copy(h_hbm.at[srcb.at[0]], rows0, gs0)

            def _pair(j2, _):
                for parity in range(2):
                    rb, gs, ss = bufs[parity]
                    ro, go, so = bufs[1 - parity]
                    j = j2 * 2 + parity

                    @pl.when(j < m)
                    def _():
                        pltpu.make_async_copy(h_hbm.at[srcb.at[0]], rb,
                                              gs).wait()

                        @pl.when(j >= 1)
                        def _():
                            pltpu.make_async_copy(
                                ro, acc_sh.at[dstb.at[0]], so).wait()

                        @pl.when(j + 1 < m)
                        def _():
                            pltpu.async_copy(h_hbm.at[srcb.at[j + 1]], ro,
                                             go)

                        _scale(rb, j)
                        pltpu.async_copy(rb, acc_sh.at[dstb.at[j]], ss,
                                         add=True)
                return 0
            lax.fori_loop(0, (m + 1) // 2, _pair, 0)

            @pl.when(m > 0)
            def _():
                par = (m - 1) % 2

                @pl.when(par == 0)
                def _():
                    pltpu.make_async_copy(rows0, acc_sh.at[dstb.at[0]],
                                          ss0).wait()

                @pl.when(par == 1)
                def _():
                    pltpu.make_async_copy(rows1, acc_sh.at[dstb.at[0]],
                                          ss1).wait()
            return 0
        lax.fori_loop(0, nsb, _sb, 0)

        plsc.subcore_barrier()

        def _cp(k, _):
            off = s * STRIPE + k * CH2
            pltpu.sync_copy(acc_sh.at[pl.ds(off, CH2)], rows0)
            pltpu.sync_copy(rows0, acc_out.at[pl.ds(c * ACC_R + off, CH2)])
            return 0
        lax.fori_loop(0, STRIPE // CH2, _cp, 0)

    return sc_rows


_sc_kernels = None


def _get_sc_kernels():
    global _sc_kernels
    if _sc_kernels is None:
        _sc_kernels = (_make_sc_ex(), _make_sc_rows())
    return _sc_kernels


def kernel(x, edge_index, batch, W1, a_s1, a_d1, b1, W2, a_s2, a_d2, b2, aw,
           ab, fc1_w, fc1_b, bn_g, bn_b, fc2_w, fc2_b):
    f32 = jnp.float32
    src = edge_index[0].astype(jnp.int32)
    dst = edge_index[1].astype(jnp.int32)
    pad = EPAD - E
    src_p = jnp.concatenate([src, jnp.zeros((pad,), jnp.int32)])
    dst_p = jnp.concatenate([dst, jnp.full((pad,), N, jnp.int32)])
    src2d = src_p.reshape(NW, KCH, CH)
    dst2d = dst_p.reshape(NW, KCH, CH)
    src64 = src_p.reshape(NW, KCH2, CH2)
    dst64 = dst_p.reshape(NW, KCH2, CH2)

    as1r = a_s1.reshape(1, D).astype(f32)
    ad1r = a_d1.reshape(1, D).astype(f32)
    as2r = a_s2.reshape(1, D).astype(f32)
    ad2r = a_d2.reshape(1, D).astype(f32)
    b1r = b1.reshape(1, D).astype(f32)
    b2r = b2.reshape(1, D).astype(f32)

    sc_ex, sc_rows = _get_sc_kernels()

    h1, s1, d1 = _tc1(x.astype(f32), W1.astype(f32), as1r, ad1r)
    as_pad = jnp.concatenate([s1.reshape(N), jnp.zeros((TAB - N,), f32)])
    ad_pad = jnp.concatenate([d1.reshape(N), jnp.zeros((TAB - N,), f32)])
    ex1, den = sc_ex(src2d, dst2d, as_pad, ad_pad)
    acc = sc_rows(h1, src64, dst64, ex1)
    acc0, acc1 = acc[:N], acc[ACC_R:ACC_R + N]
    den0 = den[:N].reshape(N, 1)
    den1 = den[ACC_R:ACC_R + N].reshape(N, 1)

    h2, s2, d2 = _tc2(acc0, acc1, den0, den1, h1, s1, d1, b1r,
                      W2.astype(f32), as2r, ad2r)
    as_pad2 = jnp.concatenate([s2.reshape(N), jnp.zeros((TAB - N,), f32)])
    ad_pad2 = jnp.concatenate([d2.reshape(N), jnp.zeros((TAB - N,), f32)])
    ex2, denx = sc_ex(src2d, dst2d, as_pad2, ad_pad2)
    accb = sc_rows(h2, src64, dst64, ex2)
    acc0b, acc1b = accb[:N], accb[ACC_R:ACC_R + N]
    den0b = denx[:N].reshape(N, 1)
    den1b = denx[ACC_R:ACC_R + N].reshape(N, 1)

    hf, s_col = _tc3a(acc0b, acc1b, den0b, den1b, h2, s2, d2, b2r,
                      aw.reshape(D, 1).astype(f32),
                      ab.reshape(1, 1).astype(f32))
    attn = _tc3b(s_col)
    batch_col = batch.astype(jnp.int32).reshape(N, 1)
    z = _tc3c(hf, attn, batch_col,
              fc1_w.astype(f32), fc1_b.reshape(1, -1).astype(f32),
              bn_g.reshape(1, -1).astype(f32), bn_b.reshape(1, -1).astype(f32),
              fc2_w.astype(f32), fc2_b.reshape(1, -1).astype(f32))
    return z
